# Initial kernel scaffold; baseline (speedup 1.0000x reference)
import jax
import jax.numpy as jnp
from jax.experimental import pallas as pl


def kernel(tensor, tensor_slice_index, probability):
    raise NotImplementedError("write your pallas kernel here")


# speedup vs baseline: 1.0252x; 1.0252x over previous
import numpy as np
import jax
import jax.numpy as jnp
from jax.experimental import pallas as pl
from jax.experimental.pallas import tpu as pltpu

_N_ROWS = 16384
_N_COLS = 2048
_BLOCK_R = 512


def _uniform_consts() -> np.ndarray:
    ks0, ks1 = np.uint32(0), np.uint32(42)
    ks2 = np.uint32(ks0 ^ ks1 ^ np.uint32(0x1BD11BDA))
    ks = [ks0, ks1, ks2]
    rot = [(13, 15, 26, 6), (17, 29, 16, 24)]

    def rotl(x, r):
        r = np.uint32(r)
        return ((x << r) | (x >> np.uint32(32 - r))).astype(np.uint32)

    x0 = np.full(_N_COLS, ks0, dtype=np.uint32)
    x1 = (np.arange(_N_COLS, dtype=np.uint32) + ks1).astype(np.uint32)
    for i in range(5):
        for r in rot[i % 2]:
            x0 = (x0 + x1).astype(np.uint32)
            x1 = rotl(x1, r)
            x1 = (x1 ^ x0).astype(np.uint32)
        x0 = (x0 + ks[(i + 1) % 3]).astype(np.uint32)
        x1 = (x1 + ks[(i + 2) % 3] + np.uint32(i + 1)).astype(np.uint32)
    bits = (x0 ^ x1).astype(np.uint32)
    fb = ((bits >> np.uint32(9)) | np.uint32(0x3F800000)).astype(np.uint32)
    u = fb.view(np.float32) - np.float32(1.0)
    return np.broadcast_to(u, (8, _N_COLS)).copy()


_U_TILE = _uniform_consts()


def _copy_flip_body(in_ref, u_ref, idx_ref, prob_ref, out_ref):
    out_ref[...] = in_ref[...]
    idx = idx_ref[0]
    i = pl.program_id(0)

    @pl.when(i == idx // _BLOCK_R)
    def _flip_row():
        r = idx % _BLOCK_R
        row = in_ref[pl.ds(r, 1), :]
        sample = (u_ref[pl.ds(0, 1), :] < prob_ref[0]).astype(jnp.float32)
        out_ref[pl.ds(r, 1), :] = jnp.abs(row - sample)


def kernel(tensor, tensor_slice_index, probability):
    idx = jnp.asarray(tensor_slice_index, jnp.int32).reshape((1,))
    prob = jnp.asarray(probability, jnp.float32).reshape((1,))
    u = jnp.asarray(_U_TILE)
    grid = _N_ROWS // _BLOCK_R
    out = pl.pallas_call(
        _copy_flip_body,
        grid=(grid,),
        in_specs=[
            pl.BlockSpec((_BLOCK_R, _N_COLS), lambda i: (i, 0)),
            pl.BlockSpec((8, _N_COLS), lambda i: (0, 0)),
            pl.BlockSpec(memory_space=pltpu.SMEM),
            pl.BlockSpec(memory_space=pltpu.SMEM),
        ],
        out_specs=pl.BlockSpec((_BLOCK_R, _N_COLS), lambda i: (i, 0)),
        out_shape=jax.ShapeDtypeStruct((_N_ROWS, _N_COLS), jnp.float32),
    )(tensor, u, idx, prob)
    return (out, tensor_slice_index)


# speedup vs baseline: 1.0441x; 1.0184x over previous
import numpy as np
import jax
import jax.numpy as jnp
from jax.experimental import pallas as pl
from jax.experimental.pallas import tpu as pltpu

_N_ROWS = 16384
_N_COLS = 2048
_BLOCK_R = 1024


def _uniform_consts() -> np.ndarray:
    ks0, ks1 = np.uint32(0), np.uint32(42)
    ks2 = np.uint32(ks0 ^ ks1 ^ np.uint32(0x1BD11BDA))
    ks = [ks0, ks1, ks2]
    rot = [(13, 15, 26, 6), (17, 29, 16, 24)]

    def rotl(x, r):
        r = np.uint32(r)
        return ((x << r) | (x >> np.uint32(32 - r))).astype(np.uint32)

    x0 = np.full(_N_COLS, ks0, dtype=np.uint32)
    x1 = (np.arange(_N_COLS, dtype=np.uint32) + ks1).astype(np.uint32)
    for i in range(5):
        for r in rot[i % 2]:
            x0 = (x0 + x1).astype(np.uint32)
            x1 = rotl(x1, r)
            x1 = (x1 ^ x0).astype(np.uint32)
        x0 = (x0 + ks[(i + 1) % 3]).astype(np.uint32)
        x1 = (x1 + ks[(i + 2) % 3] + np.uint32(i + 1)).astype(np.uint32)
    bits = (x0 ^ x1).astype(np.uint32)
    fb = ((bits >> np.uint32(9)) | np.uint32(0x3F800000)).astype(np.uint32)
    u = fb.view(np.float32) - np.float32(1.0)
    return np.broadcast_to(u, (8, _N_COLS)).copy()


_U_TILE = _uniform_consts()


def _copy_flip_body(in_ref, u_ref, idx_ref, prob_ref, out_ref):
    out_ref[...] = in_ref[...]
    idx = idx_ref[0]
    i = pl.program_id(0)

    @pl.when(i == idx // _BLOCK_R)
    def _flip_row():
        r = idx % _BLOCK_R
        row = in_ref[pl.ds(r, 1), :]
        sample = (u_ref[pl.ds(0, 1), :] < prob_ref[0]).astype(jnp.float32)
        out_ref[pl.ds(r, 1), :] = jnp.abs(row - sample)


def kernel(tensor, tensor_slice_index, probability):
    idx = jnp.asarray(tensor_slice_index, jnp.int32).reshape((1,))
    prob = jnp.asarray(probability, jnp.float32).reshape((1,))
    u = jnp.asarray(_U_TILE)
    grid = _N_ROWS // _BLOCK_R
    out = pl.pallas_call(
        _copy_flip_body,
        grid=(grid,),
        in_specs=[
            pl.BlockSpec((_BLOCK_R, _N_COLS), lambda i: (i, 0)),
            pl.BlockSpec((8, _N_COLS), lambda i: (0, 0)),
            pl.BlockSpec(memory_space=pltpu.SMEM),
            pl.BlockSpec(memory_space=pltpu.SMEM),
        ],
        out_specs=pl.BlockSpec((_BLOCK_R, _N_COLS), lambda i: (i, 0)),
        out_shape=jax.ShapeDtypeStruct((_N_ROWS, _N_COLS), jnp.float32),
    )(tensor, u, idx, prob)
    return (out, tensor_slice_index)
